# Initial kernel scaffold; baseline (speedup 1.0000x reference)
import jax
import jax.numpy as jnp
from jax.experimental import pallas as pl


def kernel(feat, ndix):
    raise NotImplementedError("write your pallas kernel here")


# speedup vs baseline: 1.5639x; 1.5639x over previous
import functools

import jax
import jax.numpy as jnp
from jax import lax
from jax.experimental import pallas as pl
from jax.experimental.pallas import tpu as pltpu
from jax.experimental.pallas import tpu_sc as plsc

NC = 2
NS = 16
L = 16
NW = NC * NS


@functools.lru_cache(maxsize=None)
def _make_sc_kernel(N, F, K, NP, B):
    RW = NP // NW
    NCH = RW // B
    FC = F // L
    mesh = plsc.VectorSubcoreMesh(core_axis_name="c", subcore_axis_name="s")

    @functools.partial(
        pl.kernel,
        out_type=jax.ShapeDtypeStruct((NP * 2 * F,), jnp.float32),
        mesh=mesh,
        scratch_types=[
            pltpu.VMEM((RW * K,), jnp.int32),
            pltpu.VMEM((2, B * K, F), jnp.float32),
            pltpu.VMEM((B * 2 * F,), jnp.float32),
            pltpu.SemaphoreType.DMA,
            pltpu.SemaphoreType.DMA,
        ],
    )
    def body(feat_hbm, ndix_hbm, out_hbm, idx_v, rows_v, out_v, sem0, sem1):
        wid = lax.axis_index("s") * NC + lax.axis_index("c")
        base_row = wid * RW
        sems = (sem0, sem1)

        pltpu.sync_copy(
            ndix_hbm.at[pl.ds(pl.multiple_of(base_row * K, 8), RW * K)], idx_v
        )

        def start_gather(g, buf):
            idx = idx_v.at[pl.ds(pl.multiple_of(g * (B * K), 8), B * K)]
            return pltpu.async_copy(feat_hbm.at[idx], rows_v.at[buf], sems[buf])

        start_gather(0, 0)

        def do_chunk(g, buf):
            start_gather((g + 1) % NCH, 1 - buf)
            pltpu.make_async_copy(
                feat_hbm.at[pl.ds(0, B * K)], rows_v.at[buf], sems[buf]
            ).wait()
            R = rows_v.at[buf]
            for r in range(B):
                first = [R[r * K, pl.ds(c * L, L)] for c in range(FC)]

                def kstep(k, carry):
                    sums, maxs = carry
                    vals = [R[r * K + k, pl.ds(c * L, L)] for c in range(FC)]
                    return (
                        [s + v for s, v in zip(sums, vals)],
                        [jnp.maximum(m, v) for m, v in zip(maxs, vals)],
                    )

                sums, maxs = lax.fori_loop(1, K, kstep, (first, first))
                for c in range(FC):
                    out_v[pl.ds(r * 2 * F + c * L, L)] = sums[c] * (1.0 / K)
                    out_v[pl.ds(r * 2 * F + F + c * L, L)] = maxs[c]
            pltpu.sync_copy(
                out_v,
                out_hbm.at[
                    pl.ds(
                        pl.multiple_of((base_row + g * B) * 2 * F, 8), B * 2 * F
                    )
                ],
            )

        def outer(gg, carry):
            for b in range(2):
                do_chunk(gg * 2 + b, b)
            return carry

        lax.fori_loop(0, NCH // 2, outer, 0)
        pltpu.make_async_copy(
            feat_hbm.at[pl.ds(0, B * K)], rows_v.at[0], sems[0]
        ).wait()

    return body


def kernel(feat, ndix):
    N, F = feat.shape
    K = ndix.shape[1]
    B = 128 // K if K <= 128 else 1
    align = NW * B * 2
    NP = ((N + align - 1) // align) * align
    ndix_flat = jnp.pad(ndix, ((0, NP - N), (0, 0))).reshape(-1)
    out = _make_sc_kernel(N, F, K, NP, B)(feat, ndix_flat)
    return out.reshape(NP, 2 * F)[:N]


# speedup vs baseline: 7.3148x; 4.6774x over previous
import functools

import jax
import jax.numpy as jnp
from jax import lax
from jax.experimental import pallas as pl
from jax.experimental.pallas import tpu as pltpu
from jax.experimental.pallas import tpu_sc as plsc

NC = 2
NS = 16
L = 16
NW = NC * NS


@functools.lru_cache(maxsize=None)
def _make_sc_kernel(N, F, K, NP, B):
    RW = NP // NW
    NCH = RW // B
    FC = F // L
    OW = B * 2 * F
    mesh = plsc.VectorSubcoreMesh(core_axis_name="c", subcore_axis_name="s")

    @functools.partial(
        pl.kernel,
        out_type=jax.ShapeDtypeStruct((NP * 2 * F,), jnp.float32),
        mesh=mesh,
        scratch_types=[
            pltpu.VMEM_SHARED((N, F), jnp.float32),
            pltpu.VMEM((RW * K,), jnp.int32),
            pltpu.VMEM((2, B * K, F), jnp.float32),
            pltpu.VMEM((2, OW), jnp.float32),
            pltpu.SemaphoreType.DMA,
            pltpu.SemaphoreType.DMA,
            pltpu.SemaphoreType.DMA,
            pltpu.SemaphoreType.DMA,
        ],
    )
    def body(feat_hbm, ndix_hbm, out_hbm, feat_sh, idx_v, rows_v, out_v,
             gsem0, gsem1, osem0, osem1):
        cid = lax.axis_index("c")
        sid = lax.axis_index("s")
        wid = sid * NC + cid
        base_row = wid * RW
        gsems = (gsem0, gsem1)
        osems = (osem0, osem1)

        @pl.when(sid == 0)
        def _():
            pltpu.sync_copy(feat_hbm, feat_sh)

        pltpu.sync_copy(
            ndix_hbm.at[pl.ds(pl.multiple_of(base_row * K, 8), RW * K)], idx_v
        )
        plsc.subcore_barrier()

        def start_gather(g, buf):
            idx = idx_v.at[pl.ds(pl.multiple_of(g * (B * K), 8), B * K)]
            pltpu.async_copy(feat_sh.at[idx], rows_v.at[buf], gsems[buf])

        def wait_gather(buf):
            pltpu.make_async_copy(
                feat_sh.at[pl.ds(0, B * K)], rows_v.at[buf], gsems[buf]
            ).wait()

        def wait_out(buf):
            pltpu.make_async_copy(
                out_v.at[buf], out_hbm.at[pl.ds(0, OW)], osems[buf]
            ).wait()

        def do_chunk(g, buf, wait_prev_out):
            start_gather((g + 1) % NCH, 1 - buf)
            wait_gather(buf)
            if wait_prev_out:
                wait_out(buf)
            R = rows_v.at[buf]
            O = out_v.at[buf]
            for r in range(B):
                first = [R[r * K, pl.ds(c * L, L)] for c in range(FC)]

                def kstep(k, carry):
                    sums, maxs = carry
                    vals = [R[r * K + k, pl.ds(c * L, L)] for c in range(FC)]
                    return (
                        [s + v for s, v in zip(sums, vals)],
                        [jnp.maximum(m, v) for m, v in zip(maxs, vals)],
                    )

                sums, maxs = lax.fori_loop(1, K, kstep, (first, first))
                for c in range(FC):
                    O[pl.ds(r * 2 * F + c * L, L)] = sums[c] * (1.0 / K)
                    O[pl.ds(r * 2 * F + F + c * L, L)] = maxs[c]
            pltpu.async_copy(
                O,
                out_hbm.at[
                    pl.ds(pl.multiple_of((base_row + g * B) * 2 * F, 8), OW)
                ],
                osems[buf],
            )

        start_gather(0, 0)
        do_chunk(0, 0, False)
        do_chunk(1, 1, False)

        def outer(gg, carry):
            for b in range(2):
                do_chunk(2 + gg * 2 + b, b, True)
            return carry

        lax.fori_loop(0, (NCH - 2) // 2, outer, 0)
        wait_out(0)
        wait_out(1)
        wait_gather(0)

    return body


def kernel(feat, ndix):
    N, F = feat.shape
    K = ndix.shape[1]
    B = 128 // K if K <= 128 else 1
    align = NW * B * 2
    NP = ((N + align - 1) // align) * align
    if NP // NW // B < 4:
        NP = 4 * NW * B
    ndix_flat = jnp.pad(ndix, ((0, NP - N), (0, 0))).reshape(-1)
    out = _make_sc_kernel(N, F, K, NP, B)(feat, ndix_flat)
    return out.reshape(NP, 2 * F)[:N]


# speedup vs baseline: 9.3469x; 1.2778x over previous
import functools

import jax
import jax.numpy as jnp
from jax import lax
from jax.experimental import pallas as pl
from jax.experimental.pallas import tpu as pltpu
from jax.experimental.pallas import tpu_sc as plsc

NC = 2
NS = 16
L = 16
NW = NC * NS


@functools.lru_cache(maxsize=None)
def _make_sc_kernel(N, F, K, RW, B):
    NCH = RW // B
    FC = F // L
    mesh = plsc.VectorSubcoreMesh(core_axis_name="c", subcore_axis_name="s")

    @functools.partial(
        pl.kernel,
        out_type=jax.ShapeDtypeStruct((N, 2 * F), jnp.float32),
        mesh=mesh,
        scratch_types=[
            pltpu.VMEM_SHARED((N, F), jnp.float32),
            pltpu.VMEM((RW * K,), jnp.int32),
            pltpu.VMEM((2, B * K, F), jnp.float32),
            pltpu.VMEM((2, B, 2 * F), jnp.float32),
            pltpu.SemaphoreType.DMA,
            pltpu.SemaphoreType.DMA,
            pltpu.SemaphoreType.DMA,
            pltpu.SemaphoreType.DMA,
        ],
    )
    def body(feat_hbm, ndix_hbm, out_hbm, feat_sh, idx_v, rows_v, out_v,
             gsem0, gsem1, osem0, osem1):
        cid = lax.axis_index("c")
        sid = lax.axis_index("s")
        wid = sid * NC + cid
        base_row = jnp.minimum(wid * RW, N - RW)
        gsems = (gsem0, gsem1)
        osems = (osem0, osem1)

        @pl.when(sid == 0)
        def _():
            pltpu.sync_copy(feat_hbm, feat_sh)

        pltpu.sync_copy(
            ndix_hbm.at[pl.ds(pl.multiple_of(base_row * K, 8), RW * K)], idx_v
        )
        plsc.subcore_barrier()

        def start_gather(g, buf):
            idx = idx_v.at[pl.ds(pl.multiple_of(g * (B * K), 8), B * K)]
            pltpu.async_copy(feat_sh.at[idx], rows_v.at[buf], gsems[buf])

        def wait_gather(buf):
            pltpu.make_async_copy(
                feat_sh.at[pl.ds(0, B * K)], rows_v.at[buf], gsems[buf]
            ).wait()

        def wait_out(buf):
            pltpu.make_async_copy(
                out_v.at[buf], out_hbm.at[pl.ds(0, B)], osems[buf]
            ).wait()

        def do_chunk(g, buf, wait_prev_out):
            start_gather((g + 1) % NCH, 1 - buf)
            wait_gather(buf)
            if wait_prev_out:
                wait_out(buf)
            R = rows_v.at[buf]
            O = out_v.at[buf]
            for r in range(B):
                first = [R[r * K, pl.ds(c * L, L)] for c in range(FC)]

                def kstep(k, carry):
                    sums, maxs = carry
                    vals = [R[r * K + k, pl.ds(c * L, L)] for c in range(FC)]
                    return (
                        [s + v for s, v in zip(sums, vals)],
                        [jnp.maximum(m, v) for m, v in zip(maxs, vals)],
                    )

                sums, maxs = lax.fori_loop(1, K, kstep, (first, first))
                for c in range(FC):
                    O[r, pl.ds(c * L, L)] = sums[c] * (1.0 / K)
                    O[r, pl.ds(F + c * L, L)] = maxs[c]
            pltpu.async_copy(
                O, out_hbm.at[pl.ds(base_row + g * B, B)], osems[buf]
            )

        start_gather(0, 0)
        do_chunk(0, 0, False)
        do_chunk(1, 1, False)

        def outer(gg, carry):
            for b in range(2):
                do_chunk(2 + gg * 2 + b, b, True)
            return carry

        lax.fori_loop(0, (NCH - 2) // 2, outer, 0)
        wait_out(0)
        wait_out(1)
        wait_gather(0)

    return body


def kernel(feat, ndix):
    N, F = feat.shape
    K = ndix.shape[1]
    B = 128 // K if K <= 128 else 1
    RW = ((N + NW - 1) // NW + 2 * B - 1) // (2 * B) * (2 * B)
    RW = max(RW, 4 * B)
    ndix_flat = ndix.reshape(-1)
    return _make_sc_kernel(N, F, K, RW, B)(feat, ndix_flat)
